# baseline (speedup 1.0000x reference)
import functools

import jax
import jax.numpy as jnp
from jax import lax
from jax.experimental import pallas as pl
from jax.experimental.pallas import tpu as pltpu
from jax.experimental.pallas import tpu_sc as plsc

N_NODES = 10000
N_EDGES = 320000
D = 128

NC = 2
NS = 16
NW = NC * NS

K = 128
STEPS = 80
EW = STEPS * K
E_PAD = EW * NW
N_PAD = 10112
ROWS_PER_TILE = N_PAD // NS


@functools.partial(
    pl.kernel,
    out_type=jax.ShapeDtypeStruct((NC, N_PAD, D), jnp.float32),
    mesh=plsc.VectorSubcoreMesh(core_axis_name="c", subcore_axis_name="s"),
    scratch_types=[
        pltpu.VMEM((STEPS, K), jnp.int32),
        pltpu.VMEM((K,), jnp.int32),
        pltpu.VMEM((K,), jnp.int32),
        pltpu.VMEM((K, D), jnp.float32),
        pltpu.VMEM((K, D), jnp.float32),
        pltpu.VMEM_SHARED((N_PAD, D), jnp.float32),
        pltpu.SemaphoreType.DMA,
        pltpu.SemaphoreType.DMA,
        pltpu.SemaphoreType.DMA,
        pltpu.SemaphoreType.DMA,
        pltpu.SemaphoreType.DMA,
        pltpu.SemaphoreType.DMA,
    ],
)
def _sc_agg(src_hbm, dst_hbm, x_hbm, out_hbm,
            idx_s, di0, di1, rows0, rows1, agg,
            sem_g0, sem_g1, sem_s0, sem_s1, sem_i0, sem_i1):
    c = lax.axis_index("c")
    s = lax.axis_index("s")
    wid = c * NS + s
    ebase = wid * EW

    pltpu.async_copy(src_hbm.at[wid], idx_s, sem_g0)

    def _zfill(r, _):
        for j in range(D // 16):
            rows0[r, pl.ds(j * 16, 16)] = jnp.zeros((16,), jnp.float32)
        return _
    lax.fori_loop(0, K, _zfill, None)
    n_full = ROWS_PER_TILE // K
    for b in range(n_full):
        pltpu.sync_copy(rows0, agg.at[pl.ds(s * ROWS_PER_TILE + b * K, K)])
    rem = ROWS_PER_TILE - n_full * K
    if rem:
        pltpu.sync_copy(rows0.at[pl.ds(0, rem)],
                        agg.at[pl.ds(s * ROWS_PER_TILE + n_full * K, rem)])
    pltpu.make_async_copy(src_hbm.at[wid], idx_s, sem_g0).wait()
    plsc.subcore_barrier()

    def _wait_g(buf, sem):
        pltpu.make_async_copy(x_hbm.at[idx_s.at[0]], buf, sem).wait()

    def _wait_s(buf, di, sem):
        pltpu.make_async_copy(buf, agg.at[di], sem).wait()

    def _wait_i(di, sem):
        pltpu.make_async_copy(dst_hbm.at[pl.ds(0, K)], di, sem).wait()

    pltpu.async_copy(dst_hbm.at[pl.ds(ebase, K)], di0, sem_i0)
    pltpu.async_copy(dst_hbm.at[pl.ds(ebase + K, K)], di1, sem_i1)
    pltpu.async_copy(x_hbm.at[idx_s.at[0]], rows0, sem_g0)

    def _pair(t, _):
        a = 2 * t
        b = a + 1
        _wait_g(rows0, sem_g0)

        @pl.when(t > 0)
        def _():
            _wait_s(rows1, di1, sem_s1)
            pltpu.async_copy(dst_hbm.at[pl.ds(ebase + b * K, K)], di1, sem_i1)
        pltpu.async_copy(x_hbm.at[idx_s.at[b]], rows1, sem_g1)
        _wait_i(di0, sem_i0)
        pltpu.async_copy(rows0, agg.at[di0], sem_s0, add=True)
        _wait_g(rows1, sem_g1)
        _wait_s(rows0, di0, sem_s0)

        @pl.when(t < STEPS // 2 - 1)
        def _():
            pltpu.async_copy(x_hbm.at[idx_s.at[a + 2]], rows0, sem_g0)
            pltpu.async_copy(dst_hbm.at[pl.ds(ebase + (a + 2) * K, K)],
                             di0, sem_i0)
        _wait_i(di1, sem_i1)
        pltpu.async_copy(rows1, agg.at[di1], sem_s1, add=True)
        return _

    lax.fori_loop(0, STEPS // 2, _pair, None)
    _wait_s(rows1, di1, sem_s1)
    plsc.subcore_barrier()

    pltpu.sync_copy(agg.at[pl.ds(s * ROWS_PER_TILE, ROWS_PER_TILE)],
                    out_hbm.at[c, pl.ds(s * ROWS_PER_TILE, ROWS_PER_TILE)])


ROWS_BLK = 1000


def _tc_dense_kernel(agg_ref, x_ref, wrel_ref, wroot_ref, b_ref, out_ref):
    a = agg_ref[0] + agg_ref[1]
    acc = jnp.dot(a, wrel_ref[...], preferred_element_type=jnp.float32)
    acc += jnp.dot(x_ref[...], wroot_ref[...], preferred_element_type=jnp.float32)
    out_ref[...] = jnp.maximum(acc + b_ref[...], 0.0)


def _tc_dense(agg2, x, wrel_t, wroot_t, b2d):
    grid = (N_NODES // ROWS_BLK,)
    return pl.pallas_call(
        _tc_dense_kernel,
        grid=grid,
        in_specs=[
            pl.BlockSpec((NC, ROWS_BLK, D), lambda i: (0, i, 0)),
            pl.BlockSpec((ROWS_BLK, D), lambda i: (i, 0)),
            pl.BlockSpec((D, D), lambda i: (0, 0)),
            pl.BlockSpec((D, D), lambda i: (0, 0)),
            pl.BlockSpec((1, D), lambda i: (0, 0)),
        ],
        out_specs=pl.BlockSpec((ROWS_BLK, D), lambda i: (i, 0)),
        out_shape=jax.ShapeDtypeStruct((N_NODES, D), jnp.float32),
    )(agg2, x, wrel_t, wroot_t, b2d)


def kernel(x, edge_index, W_rel, b_rel, W_root):
    ei = edge_index.astype(jnp.int32)
    pad = E_PAD - N_EDGES
    src = jnp.concatenate([ei[0], jnp.zeros((pad,), jnp.int32)])
    pad_dst = N_NODES + (jnp.arange(pad, dtype=jnp.int32) % (N_PAD - N_NODES))
    dst = jnp.concatenate([ei[1], pad_dst])
    src = src.reshape(NW, STEPS, K)
    agg2 = _sc_agg(src, dst, x)
    return _tc_dense(agg2, x, W_rel.T, W_root.T, b_rel[None, :])


# speedup vs baseline: 1.0673x; 1.0673x over previous
import functools

import jax
import jax.numpy as jnp
from jax import lax
from jax.experimental import pallas as pl
from jax.experimental.pallas import tpu as pltpu
from jax.experimental.pallas import tpu_sc as plsc

N_NODES = 10000
N_EDGES = 320000
D = 128

NC = 2
NS = 16
NW = NC * NS

K = 64
STEPS = 160
UNROLL = 4
EW = STEPS * K
E_PAD = EW * NW
N_PAD = 10112
ROWS_PER_TILE = N_PAD // NS


@functools.partial(
    pl.kernel,
    out_type=jax.ShapeDtypeStruct((NC, N_PAD, D), jnp.float32),
    mesh=plsc.VectorSubcoreMesh(core_axis_name="c", subcore_axis_name="s"),
    scratch_types=[
        pltpu.VMEM((EW,), jnp.int32),
        pltpu.VMEM((K,), jnp.int32),
        pltpu.VMEM((K,), jnp.int32),
        pltpu.VMEM((K,), jnp.int32),
        pltpu.VMEM((K,), jnp.int32),
        pltpu.VMEM((K, D), jnp.float32),
        pltpu.VMEM((K, D), jnp.float32),
        pltpu.VMEM((K, D), jnp.float32),
        pltpu.VMEM((K, D), jnp.float32),
        pltpu.VMEM_SHARED((N_PAD, D), jnp.float32),
        pltpu.SemaphoreType.DMA,
        pltpu.SemaphoreType.DMA,
        pltpu.SemaphoreType.DMA,
        pltpu.SemaphoreType.DMA,
        pltpu.SemaphoreType.DMA,
        pltpu.SemaphoreType.DMA,
        pltpu.SemaphoreType.DMA,
        pltpu.SemaphoreType.DMA,
        pltpu.SemaphoreType.DMA,
        pltpu.SemaphoreType.DMA,
        pltpu.SemaphoreType.DMA,
        pltpu.SemaphoreType.DMA,
    ],
)
def _sc_agg(src_hbm, dst_hbm, x_hbm, out_hbm,
            idx_s, d0, d1, d2, d3, r0, r1, r2, r3, agg,
            g0, g1, g2, g3, s0, s1, s2, s3, i0, i1, i2, i3):
    c = lax.axis_index("c")
    s = lax.axis_index("s")
    wid = c * NS + s
    ebase = wid * EW

    rows = (r0, r1, r2, r3)
    dib = (d0, d1, d2, d3)
    gsem = (g0, g1, g2, g3)
    ssem = (s0, s1, s2, s3)
    isem = (i0, i1, i2, i3)

    pltpu.async_copy(src_hbm.at[wid], idx_s, g0)

    def _zfill(r, _):
        for j in range(D // 16):
            r0[r, pl.ds(j * 16, 16)] = jnp.zeros((16,), jnp.float32)
        return _
    lax.fori_loop(0, K, _zfill, None)
    n_full = ROWS_PER_TILE // K
    for b in range(n_full):
        pltpu.sync_copy(r0, agg.at[pl.ds(s * ROWS_PER_TILE + b * K, K)])
    rem = ROWS_PER_TILE - n_full * K
    if rem:
        pltpu.sync_copy(r0.at[pl.ds(0, rem)],
                        agg.at[pl.ds(s * ROWS_PER_TILE + n_full * K, rem)])
    pltpu.make_async_copy(src_hbm.at[wid], idx_s, g0).wait()
    plsc.subcore_barrier()

    def _gather(j, u):
        pltpu.async_copy(x_hbm.at[idx_s.at[pl.ds(j * K, K)]], rows[u], gsem[u])

    def _load_di(j, u):
        pltpu.async_copy(dst_hbm.at[pl.ds(ebase + j * K, K)], dib[u], isem[u])

    def _scatter(j, u):
        pltpu.async_copy(rows[u], agg.at[dib[u]], ssem[u], add=True)

    def _wait_g(u):
        pltpu.make_async_copy(x_hbm.at[idx_s.at[pl.ds(0, K)]], rows[u],
                              gsem[u]).wait()

    def _wait_s(u):
        pltpu.make_async_copy(rows[u], agg.at[dib[u]], ssem[u]).wait()

    def _wait_i(u):
        pltpu.make_async_copy(dst_hbm.at[pl.ds(0, K)], dib[u], isem[u]).wait()

    for u in range(UNROLL - 1):
        _gather(u, u)
        _load_di(u, u)

    NIT = STEPS // UNROLL

    def _body(t, _):
        j0 = t * UNROLL
        for u in range(UNROLL):
            j = j0 + u
            _wait_g(u)
            if u == 0:
                @pl.when(t > 0)
                def _():
                    _wait_s(UNROLL - 1)
            else:
                _wait_s(u - 1)
            nxt = (u + UNROLL - 1) % UNROLL

            @pl.when(j + UNROLL - 1 < STEPS)
            def _():
                _gather(j + UNROLL - 1, nxt)
                _load_di(j + UNROLL - 1, nxt)
            _wait_i(u)
            _scatter(j, u)
        return _

    lax.fori_loop(0, NIT, _body, None)
    _wait_s(UNROLL - 1)
    plsc.subcore_barrier()

    pltpu.sync_copy(agg.at[pl.ds(s * ROWS_PER_TILE, ROWS_PER_TILE)],
                    out_hbm.at[c, pl.ds(s * ROWS_PER_TILE, ROWS_PER_TILE)])


ROWS_BLK = 1000


def _tc_dense_kernel(agg_ref, x_ref, wrel_ref, wroot_ref, b_ref, out_ref):
    a = agg_ref[0] + agg_ref[1]
    acc = jnp.dot(a, wrel_ref[...], preferred_element_type=jnp.float32)
    acc += jnp.dot(x_ref[...], wroot_ref[...], preferred_element_type=jnp.float32)
    out_ref[...] = jnp.maximum(acc + b_ref[...], 0.0)


def _tc_dense(agg2, x, wrel_t, wroot_t, b2d):
    grid = (N_NODES // ROWS_BLK,)
    return pl.pallas_call(
        _tc_dense_kernel,
        grid=grid,
        in_specs=[
            pl.BlockSpec((NC, ROWS_BLK, D), lambda i: (0, i, 0)),
            pl.BlockSpec((ROWS_BLK, D), lambda i: (i, 0)),
            pl.BlockSpec((D, D), lambda i: (0, 0)),
            pl.BlockSpec((D, D), lambda i: (0, 0)),
            pl.BlockSpec((1, D), lambda i: (0, 0)),
        ],
        out_specs=pl.BlockSpec((ROWS_BLK, D), lambda i: (i, 0)),
        out_shape=jax.ShapeDtypeStruct((N_NODES, D), jnp.float32),
    )(agg2, x, wrel_t, wroot_t, b2d)


def kernel(x, edge_index, W_rel, b_rel, W_root):
    ei = edge_index.astype(jnp.int32)
    pad = E_PAD - N_EDGES
    src = jnp.concatenate([ei[0], jnp.zeros((pad,), jnp.int32)])
    pad_dst = N_NODES + (jnp.arange(pad, dtype=jnp.int32) % (N_PAD - N_NODES))
    dst = jnp.concatenate([ei[1], pad_dst])
    src = src.reshape(NW, EW)
    agg2 = _sc_agg(src, dst, x)
    return _tc_dense(agg2, x, W_rel.T, W_root.T, b_rel[None, :])


# speedup vs baseline: 2.1520x; 2.0164x over previous
import functools

import jax
import jax.numpy as jnp
from jax import lax
from jax.experimental import pallas as pl
from jax.experimental.pallas import tpu as pltpu
from jax.experimental.pallas import tpu_sc as plsc

N_NODES = 10000
N_EDGES = 320000
D = 128

NC = 2
NS = 16
NW = NC * NS

K = 64
STEPS = 160
UNROLL = 4
EW = STEPS * K
E_PAD = EW * NW
N_PAD = 10112
ROWS_PER_TILE = N_PAD // NS
XRPT = 624
XRPT_LAST = N_NODES - (NS - 1) * XRPT


@functools.partial(
    pl.kernel,
    out_type=jax.ShapeDtypeStruct((NW, EW, D), jnp.float32),
    mesh=plsc.VectorSubcoreMesh(core_axis_name="c", subcore_axis_name="s"),
    scratch_types=[
        pltpu.VMEM((EW,), jnp.int32),
        pltpu.VMEM((K, D), jnp.float32),
        pltpu.VMEM((K, D), jnp.float32),
        pltpu.VMEM((K, D), jnp.float32),
        pltpu.VMEM((K, D), jnp.float32),
        pltpu.VMEM_SHARED((N_NODES, D), jnp.float32),
        pltpu.SemaphoreType.DMA,
        pltpu.SemaphoreType.DMA,
        pltpu.SemaphoreType.DMA,
        pltpu.SemaphoreType.DMA,
        pltpu.SemaphoreType.DMA,
        pltpu.SemaphoreType.DMA,
        pltpu.SemaphoreType.DMA,
        pltpu.SemaphoreType.DMA,
    ],
)
def _sc_expand(src_hbm, x_hbm, msg_hbm,
               idx_s, r0, r1, r2, r3, xs,
               g0, g1, g2, g3, w0, w1, w2, w3):
    c = lax.axis_index("c")
    s = lax.axis_index("s")
    wid = c * NS + s

    rows = (r0, r1, r2, r3)
    gsem = (g0, g1, g2, g3)
    wsem = (w0, w1, w2, w3)

    pltpu.async_copy(src_hbm.at[wid], idx_s, g0)

    @pl.when(s < NS - 1)
    def _():
        pltpu.async_copy(x_hbm.at[pl.ds(s * XRPT, XRPT)],
                         xs.at[pl.ds(s * XRPT, XRPT)], g1)
        pltpu.make_async_copy(x_hbm.at[pl.ds(0, XRPT)],
                              xs.at[pl.ds(0, XRPT)], g1).wait()

    @pl.when(s == NS - 1)
    def _():
        pltpu.async_copy(x_hbm.at[pl.ds((NS - 1) * XRPT, XRPT_LAST)],
                         xs.at[pl.ds((NS - 1) * XRPT, XRPT_LAST)], g1)
        pltpu.make_async_copy(
            x_hbm.at[pl.ds(0, XRPT_LAST)],
            xs.at[pl.ds(0, XRPT_LAST)], g1).wait()
    pltpu.make_async_copy(src_hbm.at[wid], idx_s, g0).wait()
    plsc.subcore_barrier()

    def _gather(j, u):
        pltpu.async_copy(xs.at[idx_s.at[pl.ds(j * K, K)]], rows[u], gsem[u])

    def _write(j, u):
        pltpu.async_copy(rows[u], msg_hbm.at[wid, pl.ds(j * K, K)], wsem[u])

    def _wait_g(u):
        pltpu.make_async_copy(xs.at[idx_s.at[pl.ds(0, K)]], rows[u],
                              gsem[u]).wait()

    def _wait_w(u):
        pltpu.make_async_copy(rows[u], msg_hbm.at[wid, pl.ds(0, K)],
                              wsem[u]).wait()

    for u in range(UNROLL - 1):
        _gather(u, u)

    NIT = STEPS // UNROLL

    def _body(t, _):
        j0 = t * UNROLL
        for u in range(UNROLL):
            j = j0 + u
            _wait_g(u)
            if u == 0:
                @pl.when(t > 0)
                def _():
                    _wait_w(UNROLL - 1)
            else:
                _wait_w(u - 1)
            _write(j, u)

            @pl.when(j + UNROLL - 1 < STEPS)
            def _():
                _gather(j + UNROLL - 1, (u + UNROLL - 1) % UNROLL)
        return _

    lax.fori_loop(0, NIT, _body, None)
    _wait_w(UNROLL - 1)


@functools.partial(
    pl.kernel,
    out_type=jax.ShapeDtypeStruct((NC, N_PAD, D), jnp.float32),
    mesh=plsc.VectorSubcoreMesh(core_axis_name="c", subcore_axis_name="s"),
    scratch_types=[
        pltpu.VMEM((EW,), jnp.int32),
        pltpu.VMEM((K, D), jnp.float32),
        pltpu.VMEM((K, D), jnp.float32),
        pltpu.VMEM((K, D), jnp.float32),
        pltpu.VMEM((K, D), jnp.float32),
        pltpu.VMEM_SHARED((N_PAD, D), jnp.float32),
        pltpu.SemaphoreType.DMA,
        pltpu.SemaphoreType.DMA,
        pltpu.SemaphoreType.DMA,
        pltpu.SemaphoreType.DMA,
        pltpu.SemaphoreType.DMA,
        pltpu.SemaphoreType.DMA,
        pltpu.SemaphoreType.DMA,
        pltpu.SemaphoreType.DMA,
    ],
)
def _sc_reduce(dst_hbm, msg_hbm, out_hbm,
               idx_d, r0, r1, r2, r3, agg,
               g0, g1, g2, g3, s0, s1, s2, s3):
    c = lax.axis_index("c")
    s = lax.axis_index("s")
    wid = c * NS + s

    rows = (r0, r1, r2, r3)
    gsem = (g0, g1, g2, g3)
    ssem = (s0, s1, s2, s3)

    pltpu.async_copy(dst_hbm.at[wid], idx_d, g0)

    def _zfill(r, _):
        for j in range(D // 16):
            r0[r, pl.ds(j * 16, 16)] = jnp.zeros((16,), jnp.float32)
        return _
    lax.fori_loop(0, K, _zfill, None)
    n_full = ROWS_PER_TILE // K
    for b in range(n_full):
        pltpu.sync_copy(r0, agg.at[pl.ds(s * ROWS_PER_TILE + b * K, K)])
    rem = ROWS_PER_TILE - n_full * K
    if rem:
        pltpu.sync_copy(r0.at[pl.ds(0, rem)],
                        agg.at[pl.ds(s * ROWS_PER_TILE + n_full * K, rem)])
    pltpu.make_async_copy(dst_hbm.at[wid], idx_d, g0).wait()
    plsc.subcore_barrier()

    def _load(j, u):
        pltpu.async_copy(msg_hbm.at[wid, pl.ds(j * K, K)], rows[u], gsem[u])

    def _scatter(j, u):
        pltpu.async_copy(rows[u], agg.at[idx_d.at[pl.ds(j * K, K)]],
                         ssem[u], add=True)

    def _wait_l(u):
        pltpu.make_async_copy(msg_hbm.at[wid, pl.ds(0, K)], rows[u],
                              gsem[u]).wait()

    def _wait_s(u):
        pltpu.make_async_copy(rows[u], agg.at[idx_d.at[pl.ds(0, K)]],
                              ssem[u]).wait()

    for u in range(UNROLL - 1):
        _load(u, u)

    NIT = STEPS // UNROLL

    def _body(t, _):
        j0 = t * UNROLL
        for u in range(UNROLL):
            j = j0 + u
            _wait_l(u)
            if u == 0:
                @pl.when(t > 0)
                def _():
                    _wait_s(UNROLL - 1)
            else:
                _wait_s(u - 1)
            _scatter(j, u)

            @pl.when(j + UNROLL - 1 < STEPS)
            def _():
                _load(j + UNROLL - 1, (u + UNROLL - 1) % UNROLL)
        return _

    lax.fori_loop(0, NIT, _body, None)
    _wait_s(UNROLL - 1)
    plsc.subcore_barrier()

    pltpu.sync_copy(agg.at[pl.ds(s * ROWS_PER_TILE, ROWS_PER_TILE)],
                    out_hbm.at[c, pl.ds(s * ROWS_PER_TILE, ROWS_PER_TILE)])


ROWS_BLK = 1000


def _tc_dense_kernel(agg_ref, x_ref, wrel_ref, wroot_ref, b_ref, out_ref):
    a = agg_ref[0] + agg_ref[1]
    acc = jnp.dot(a, wrel_ref[...], preferred_element_type=jnp.float32)
    acc += jnp.dot(x_ref[...], wroot_ref[...], preferred_element_type=jnp.float32)
    out_ref[...] = jnp.maximum(acc + b_ref[...], 0.0)


def _tc_dense(agg2, x, wrel_t, wroot_t, b2d):
    grid = (N_NODES // ROWS_BLK,)
    return pl.pallas_call(
        _tc_dense_kernel,
        grid=grid,
        in_specs=[
            pl.BlockSpec((NC, ROWS_BLK, D), lambda i: (0, i, 0)),
            pl.BlockSpec((ROWS_BLK, D), lambda i: (i, 0)),
            pl.BlockSpec((D, D), lambda i: (0, 0)),
            pl.BlockSpec((D, D), lambda i: (0, 0)),
            pl.BlockSpec((1, D), lambda i: (0, 0)),
        ],
        out_specs=pl.BlockSpec((ROWS_BLK, D), lambda i: (i, 0)),
        out_shape=jax.ShapeDtypeStruct((N_NODES, D), jnp.float32),
    )(agg2, x, wrel_t, wroot_t, b2d)


def kernel(x, edge_index, W_rel, b_rel, W_root):
    ei = edge_index.astype(jnp.int32)
    pad = E_PAD - N_EDGES
    src = jnp.concatenate([ei[0], jnp.zeros((pad,), jnp.int32)])
    pad_dst = N_NODES + (jnp.arange(pad, dtype=jnp.int32) % (N_PAD - N_NODES))
    dst = jnp.concatenate([ei[1], pad_dst])
    src = src.reshape(NW, EW)
    dst = dst.reshape(NW, EW)
    msgs = _sc_expand(src, x)
    agg2 = _sc_reduce(dst, msgs)
    return _tc_dense(agg2, x, W_rel.T, W_root.T, b_rel[None, :])


# speedup vs baseline: 2.1851x; 1.0154x over previous
import functools

import jax
import jax.numpy as jnp
from jax import lax
from jax.experimental import pallas as pl
from jax.experimental.pallas import tpu as pltpu
from jax.experimental.pallas import tpu_sc as plsc

N_NODES = 10000
N_EDGES = 320000
D = 128

NC = 2
NS = 16
NW = NC * NS

K = 64
STEPS = 160
UNROLL = 4
EW = STEPS * K
E_PAD = EW * NW
N_PAD = 10112
ROWS_PER_TILE = N_PAD // NS
XRPT = 624
XRPT_LAST = N_NODES - (NS - 1) * XRPT


@functools.partial(
    pl.kernel,
    out_type=jax.ShapeDtypeStruct((NW, EW, D), jnp.float32),
    mesh=plsc.VectorSubcoreMesh(core_axis_name="c", subcore_axis_name="s"),
    scratch_types=[
        pltpu.VMEM((EW,), jnp.int32),
        pltpu.VMEM((K, D), jnp.float32),
        pltpu.VMEM((K, D), jnp.float32),
        pltpu.VMEM((K, D), jnp.float32),
        pltpu.VMEM((K, D), jnp.float32),
        pltpu.VMEM_SHARED((N_NODES, D), jnp.float32),
        pltpu.SemaphoreType.DMA,
        pltpu.SemaphoreType.DMA,
        pltpu.SemaphoreType.DMA,
        pltpu.SemaphoreType.DMA,
        pltpu.SemaphoreType.DMA,
        pltpu.SemaphoreType.DMA,
        pltpu.SemaphoreType.DMA,
        pltpu.SemaphoreType.DMA,
    ],
)
def _sc_expand(src_hbm, x_hbm, msg_hbm,
               idx_s, r0, r1, r2, r3, xs,
               g0, g1, g2, g3, w0, w1, w2, w3):
    c = lax.axis_index("c")
    s = lax.axis_index("s")
    wid = c * NS + s

    rows = (r0, r1, r2, r3)
    gsem = (g0, g1, g2, g3)
    wsem = (w0, w1, w2, w3)

    pltpu.async_copy(src_hbm.at[wid], idx_s, g0)

    @pl.when(s < NS - 1)
    def _():
        pltpu.async_copy(x_hbm.at[pl.ds(s * XRPT, XRPT)],
                         xs.at[pl.ds(s * XRPT, XRPT)], g1)
        pltpu.make_async_copy(x_hbm.at[pl.ds(0, XRPT)],
                              xs.at[pl.ds(0, XRPT)], g1).wait()

    @pl.when(s == NS - 1)
    def _():
        pltpu.async_copy(x_hbm.at[pl.ds((NS - 1) * XRPT, XRPT_LAST)],
                         xs.at[pl.ds((NS - 1) * XRPT, XRPT_LAST)], g1)
        pltpu.make_async_copy(
            x_hbm.at[pl.ds(0, XRPT_LAST)],
            xs.at[pl.ds(0, XRPT_LAST)], g1).wait()
    pltpu.make_async_copy(src_hbm.at[wid], idx_s, g0).wait()
    plsc.subcore_barrier()

    def _gather(j, u):
        pltpu.async_copy(xs.at[idx_s.at[pl.ds(j * K, K)]], rows[u], gsem[u])

    def _write(j, u):
        pltpu.async_copy(rows[u], msg_hbm.at[wid, pl.ds(j * K, K)], wsem[u])

    def _wait_g(u):
        pltpu.make_async_copy(xs.at[idx_s.at[pl.ds(0, K)]], rows[u],
                              gsem[u]).wait()

    def _wait_w(u):
        pltpu.make_async_copy(rows[u], msg_hbm.at[wid, pl.ds(0, K)],
                              wsem[u]).wait()

    for u in range(UNROLL - 1):
        _gather(u, u)

    NIT = STEPS // UNROLL

    def _body(t, _):
        j0 = t * UNROLL
        for u in range(UNROLL):
            j = j0 + u
            _wait_g(u)
            if u == 0:
                @pl.when(t > 0)
                def _():
                    _wait_w(UNROLL - 1)
            else:
                _wait_w(u - 1)
            _write(j, u)

            @pl.when(j + UNROLL - 1 < STEPS)
            def _():
                _gather(j + UNROLL - 1, (u + UNROLL - 1) % UNROLL)
        return _

    lax.fori_loop(0, NIT, _body, None)
    _wait_w(UNROLL - 1)


@functools.partial(
    pl.kernel,
    out_type=jax.ShapeDtypeStruct((NC, N_PAD, D), jnp.float32),
    mesh=plsc.VectorSubcoreMesh(core_axis_name="c", subcore_axis_name="s"),
    scratch_types=[
        pltpu.VMEM((EW,), jnp.int32),
        pltpu.VMEM((K, D), jnp.float32),
        pltpu.VMEM((K, D), jnp.float32),
        pltpu.VMEM((K, D), jnp.float32),
        pltpu.VMEM((K, D), jnp.float32),
        pltpu.VMEM_SHARED((N_PAD, D), jnp.float32),
        pltpu.SemaphoreType.DMA,
        pltpu.SemaphoreType.DMA,
        pltpu.SemaphoreType.DMA,
        pltpu.SemaphoreType.DMA,
        pltpu.SemaphoreType.DMA,
        pltpu.SemaphoreType.DMA,
        pltpu.SemaphoreType.DMA,
        pltpu.SemaphoreType.DMA,
    ],
)
def _sc_reduce(dst_hbm, msg_hbm, out_hbm,
               idx_d, r0, r1, r2, r3, agg,
               g0, g1, g2, g3, s0, s1, s2, s3):
    c = lax.axis_index("c")
    s = lax.axis_index("s")
    wid = c * NS + s

    rows = (r0, r1, r2, r3)
    gsem = (g0, g1, g2, g3)
    ssem = (s0, s1, s2, s3)

    pltpu.async_copy(dst_hbm.at[wid], idx_d, g0)

    def _zfill(r, _):
        for j in range(D // 16):
            r0[r, pl.ds(j * 16, 16)] = jnp.zeros((16,), jnp.float32)
        return _
    lax.fori_loop(0, K, _zfill, None)
    n_full = ROWS_PER_TILE // K
    for b in range(n_full):
        pltpu.sync_copy(r0, agg.at[pl.ds(s * ROWS_PER_TILE + b * K, K)])
    rem = ROWS_PER_TILE - n_full * K
    if rem:
        pltpu.sync_copy(r0.at[pl.ds(0, rem)],
                        agg.at[pl.ds(s * ROWS_PER_TILE + n_full * K, rem)])
    pltpu.make_async_copy(dst_hbm.at[wid], idx_d, g0).wait()
    plsc.subcore_barrier()

    def _load(j, u):
        pltpu.async_copy(msg_hbm.at[wid, pl.ds(j * K, K)], rows[u], gsem[u])

    def _scatter(j, u):
        pltpu.async_copy(rows[u], agg.at[idx_d.at[pl.ds(j * K, K)]],
                         ssem[u], add=True)

    def _wait_l(u):
        pltpu.make_async_copy(msg_hbm.at[wid, pl.ds(0, K)], rows[u],
                              gsem[u]).wait()

    def _wait_s(u):
        pltpu.make_async_copy(rows[u], agg.at[idx_d.at[pl.ds(0, K)]],
                              ssem[u]).wait()

    for u in range(UNROLL - 1):
        _load(u, u)

    NIT = STEPS // UNROLL

    def _body(t, _):
        j0 = t * UNROLL
        for u in range(UNROLL):
            j = j0 + u
            _wait_l(u)
            if u == 0:
                @pl.when(t > 0)
                def _():
                    _wait_s(UNROLL - 1)
            else:
                _wait_s(u - 1)
            _scatter(j, u)

            @pl.when(j + UNROLL - 1 < STEPS)
            def _():
                _load(j + UNROLL - 1, (u + UNROLL - 1) % UNROLL)
        return _

    lax.fori_loop(0, NIT, _body, None)
    _wait_s(UNROLL - 1)
    plsc.subcore_barrier()

    pltpu.sync_copy(agg.at[pl.ds(s * ROWS_PER_TILE, ROWS_PER_TILE)],
                    out_hbm.at[c, pl.ds(s * ROWS_PER_TILE, ROWS_PER_TILE)])


ROWS_BLK = 2000


def _tc_dense_kernel(agg_ref, x_ref, wrel_ref, wroot_ref, b_ref, out_ref):
    a = agg_ref[0] + agg_ref[1]
    acc = jnp.dot(a, wrel_ref[...], preferred_element_type=jnp.float32)
    acc += jnp.dot(x_ref[...], wroot_ref[...], preferred_element_type=jnp.float32)
    out_ref[...] = jnp.maximum(acc + b_ref[...], 0.0)


def _tc_dense(agg2, x, wrel_t, wroot_t, b2d):
    grid = (N_NODES // ROWS_BLK,)
    return pl.pallas_call(
        _tc_dense_kernel,
        grid=grid,
        in_specs=[
            pl.BlockSpec((NC, ROWS_BLK, D), lambda i: (0, i, 0)),
            pl.BlockSpec((ROWS_BLK, D), lambda i: (i, 0)),
            pl.BlockSpec((D, D), lambda i: (0, 0)),
            pl.BlockSpec((D, D), lambda i: (0, 0)),
            pl.BlockSpec((1, D), lambda i: (0, 0)),
        ],
        out_specs=pl.BlockSpec((ROWS_BLK, D), lambda i: (i, 0)),
        out_shape=jax.ShapeDtypeStruct((N_NODES, D), jnp.float32),
    )(agg2, x, wrel_t, wroot_t, b2d)


def kernel(x, edge_index, W_rel, b_rel, W_root):
    ei = edge_index.astype(jnp.int32)
    pad = E_PAD - N_EDGES
    src = jnp.concatenate([ei[0], jnp.zeros((pad,), jnp.int32)])
    pad_dst = N_NODES + (jnp.arange(pad, dtype=jnp.int32) % (N_PAD - N_NODES))
    dst = jnp.concatenate([ei[1], pad_dst])
    src = src.reshape(NW, EW)
    dst = dst.reshape(NW, EW)
    msgs = _sc_expand(src, x)
    agg2 = _sc_reduce(dst, msgs)
    return _tc_dense(agg2, x, W_rel.T, W_root.T, b_rel[None, :])
